# baseline (speedup 1.0000x reference)
import functools

import jax
import jax.numpy as jnp
from jax import lax
from jax.experimental import pallas as pl
from jax.experimental.pallas import tpu as pltpu
from jax.experimental.pallas import tpu_sc as plsc

N = 10000
E = 320000
D = 128
H = 64
K = 48
NSUB = 16
EPT = E // NSUB
NCH = EPT // K
TAILK = EPT - NCH * K
RPT = 624
TAIL = N - NSUB * RPT

RN = 1000
RE = 2000


def _node_mm_body(h_ref, w_ref, b_ref, o_ref):
    o_ref[...] = (
        jnp.dot(h_ref[...], w_ref[...], preferred_element_type=jnp.float32)
        + b_ref[...]
    )


def _node_mm(h, w, b):
    return pl.pallas_call(
        _node_mm_body,
        grid=(N // RN,),
        in_specs=[
            pl.BlockSpec((RN, D), lambda i: (i, 0)),
            pl.BlockSpec((D, 7 * D), lambda i: (0, 0)),
            pl.BlockSpec((1, 7 * D), lambda i: (0, 0)),
        ],
        out_specs=pl.BlockSpec((RN, 7 * D), lambda i: (i, 0)),
        out_shape=jax.ShapeDtypeStruct((N, 7 * D), jnp.float32),
    )(h, w, b)


def _edge_mm_body(x_ref, w_ref, b_ref, o_ref):
    r = (
        jnp.dot(x_ref[...], w_ref[...], preferred_element_type=jnp.float32)
        + b_ref[...]
    )
    o_ref[0] = r[:, :H]
    o_ref[1] = r[:, H:]


def _edge_mm(x, w, b):
    return pl.pallas_call(
        _edge_mm_body,
        grid=(E // RE,),
        in_specs=[
            pl.BlockSpec((RE, D), lambda i: (i, 0)),
            pl.BlockSpec((D, D), lambda i: (0, 0)),
            pl.BlockSpec((1, D), lambda i: (0, 0)),
        ],
        out_specs=pl.BlockSpec((2, RE, H), lambda i: (0, i, 0)),
        out_shape=jax.ShapeDtypeStruct((2, E, H), jnp.float32),
    )(x, w, b)


def _fin_a_body(a1_ref, flo_ref, fhi_ref, blo_ref, bhi_ref,
                hpre_ref, stats_ref, ssum, ssq):
    i = pl.program_id(0)
    nf = jnp.concatenate([flo_ref[:, :H], fhi_ref[:, :H]], axis=1)
    df = jnp.concatenate([flo_ref[:, H:], fhi_ref[:, H:]], axis=1)
    nb = jnp.concatenate([blo_ref[:, :H], bhi_ref[:, :H]], axis=1)
    db = jnp.concatenate([blo_ref[:, H:], bhi_ref[:, H:]], axis=1)
    hp = a1_ref[...] + nf / (df + 1e-6) + nb / (db + 1e-6)
    hpre_ref[...] = hp

    @pl.when(i == 0)
    def _():
        ssum[...] = jnp.zeros_like(ssum)
        ssq[...] = jnp.zeros_like(ssq)

    ssum[...] += jnp.sum(hp, axis=0, keepdims=True)
    ssq[...] += jnp.sum(hp * hp, axis=0, keepdims=True)
    stats_ref[0:1] = ssum[...]
    stats_ref[1:2] = ssq[...]


def _fin_a(a1, accf, accb):
    nb_ = N // RN
    return pl.pallas_call(
        _fin_a_body,
        grid=(nb_,),
        in_specs=[
            pl.BlockSpec((RN, D), lambda i: (i, 0)),
            pl.BlockSpec((RN, D), lambda i: (i, 0)),
            pl.BlockSpec((RN, D), lambda i: (nb_ + i, 0)),
            pl.BlockSpec((RN, D), lambda i: (i, 0)),
            pl.BlockSpec((RN, D), lambda i: (nb_ + i, 0)),
        ],
        out_specs=[
            pl.BlockSpec((RN, D), lambda i: (i, 0)),
            pl.BlockSpec((2, D), lambda i: (0, 0)),
        ],
        out_shape=[
            jax.ShapeDtypeStruct((N, D), jnp.float32),
            jax.ShapeDtypeStruct((2, D), jnp.float32),
        ],
        scratch_shapes=[
            pltpu.VMEM((1, D), jnp.float32),
            pltpu.VMEM((1, D), jnp.float32),
        ],
    )(a1, accf, accf, accb, accb)


def _fin_b_body(hpre_ref, stats_ref, h_ref, g_ref, bb_ref, o_ref):
    m = stats_ref[0:1] * (1.0 / N)
    v = stats_ref[1:2] * (1.0 / N) - m * m
    y = g_ref[...] * (hpre_ref[...] - m) / jnp.sqrt(v + 1e-5) + bb_ref[...]
    o_ref[...] = jnp.maximum(y, 0.0) + h_ref[...]


def _fin_b(hpre, stats, h, g, b):
    return pl.pallas_call(
        _fin_b_body,
        grid=(N // RN,),
        in_specs=[
            pl.BlockSpec((RN, D), lambda i: (i, 0)),
            pl.BlockSpec((2, D), lambda i: (0, 0)),
            pl.BlockSpec((RN, D), lambda i: (i, 0)),
            pl.BlockSpec((1, D), lambda i: (0, 0)),
            pl.BlockSpec((1, D), lambda i: (0, 0)),
        ],
        out_specs=pl.BlockSpec((RN, D), lambda i: (i, 0)),
        out_shape=jax.ShapeDtypeStruct((N, D), jnp.float32),
    )(hpre, stats, h, g, b)


def _sc_edge_body(src_hbm, dst_hbm, tf_hbm, b2_hbm, tb_hbm, c2_hbm,
                  b3_hbm, c3_hbm, zeros_hbm,
                  outf_hbm, outb_hbm,
                  av0, av1, bv0, bv1, ga0, ga1, gbv0, gbv1, gsc,
                  avt, bvt, gat,
                  trow0, trow1, srow0, srow1, erow0, erow1,
                  contrib, acc_sh,
                  sem_ia0, sem_ia1, sem_ib0, sem_ib1,
                  sem_a0, sem_a1, sem_b0, sem_b1, sem_e0, sem_e1, sem_s):
    av = (av0, av1)
    bv = (bv0, bv1)
    ga = (ga0, ga1)
    gbv = (gbv0, gbv1)
    trow = (trow0, trow1)
    srow = (srow0, srow1)
    erow = (erow0, erow1)
    sem_ia = (sem_ia0, sem_ia1)
    sem_ib = (sem_ib0, sem_ib1)
    sem_a = (sem_a0, sem_a1)
    sem_b = (sem_b0, sem_b1)
    sem_e = (sem_e0, sem_e1)
    c = lax.axis_index("c")
    s = lax.axis_index("s")
    ebase = s * EPT
    rbase = s * RPT

    def run_phase(a_hbm, b_hbm, prim_hbm, sec_hbm, er_hbm, out_hbm):
        pltpu.sync_copy(zeros_hbm.at[pl.ds(0, RPT)], acc_sh.at[pl.ds(rbase, RPT)])

        @pl.when(s == NSUB - 1)
        def _():
            pltpu.sync_copy(zeros_hbm.at[pl.ds(0, TAIL)],
                            acc_sh.at[pl.ds(NSUB * RPT, TAIL)])

        plsc.subcore_barrier()

        def issue_idx(i, slot):
            base = ebase + i * K
            pltpu.async_copy(a_hbm.at[pl.ds(base, K)], av[slot], sem_ia[slot])
            pltpu.async_copy(b_hbm.at[pl.ds(base, K)], bv[slot], sem_ib[slot])

        def wait_idx(slot):
            pltpu.make_async_copy(a_hbm.at[pl.ds(0, K)], av[slot],
                                  sem_ia[slot]).wait()
            pltpu.make_async_copy(b_hbm.at[pl.ds(0, K)], bv[slot],
                                  sem_ib[slot]).wait()

        def transform(slot):
            for j in range(K // 16):
                ga[slot][pl.ds(j * 16, 16)] = (
                    av[slot][pl.ds(j * 16, 16)] * 2 + c)
                gbv[slot][pl.ds(j * 16, 16)] = bv[slot][pl.ds(j * 16, 16)]

        def issue_gathers(i, slot):
            base = ebase + i * K
            pltpu.async_copy(prim_hbm.at[ga[slot]], trow[slot], sem_a[slot])
            pltpu.async_copy(sec_hbm.at[gbv[slot]], srow[slot], sem_b[slot])
            pltpu.async_copy(er_hbm.at[pl.ds(c * E + base, K)],
                             erow[slot], sem_e[slot])

        def wait_gathers(slot):
            pltpu.make_async_copy(prim_hbm.at[ga[slot]], trow[slot],
                                  sem_a[slot]).wait()
            pltpu.make_async_copy(sec_hbm.at[gbv[slot]], srow[slot],
                                  sem_b[slot]).wait()
            pltpu.make_async_copy(er_hbm.at[pl.ds(0, K)], erow[slot],
                                  sem_e[slot]).wait()

        def wait_scatter():
            pltpu.make_async_copy(contrib, acc_sh.at[gsc], sem_s).wait()

        def compute(slot, nk, cbuf, tr, sr, er):
            @plsc.parallel_loop(0, nk, unroll=4)
            def edge(e):
                for f in range(4):
                    b1 = tr[e, pl.ds(16 * f, 16)]
                    a2 = tr[e, pl.ds(H + 16 * f, 16)]
                    b2 = sr[e, pl.ds(c * H + 16 * f, 16)]
                    b3 = er[e, pl.ds(16 * f, 16)]
                    xn = b1 + b2 + b3
                    sg = 1.0 / (1.0 + jnp.exp(xn))
                    cbuf[e, pl.ds(16 * f, 16)] = a2 * sg
                    cbuf[e, pl.ds(H + 16 * f, 16)] = sg

        pltpu.sync_copy(a_hbm.at[pl.ds(ebase, K)], av[0])
        pltpu.sync_copy(b_hbm.at[pl.ds(ebase, K)], bv[0])
        transform(0)
        issue_gathers(0, 0)
        issue_idx(1, 1)

        def pair(j, carry):
            for p in (0, 1):
                i = 2 * j + p
                q = 1 - p

                @pl.when(i + 1 < NCH)
                def _():
                    wait_idx(q)
                    transform(q)
                    issue_gathers(i + 1, q)

                @pl.when(i + 2 < NCH)
                def _():
                    issue_idx(i + 2, p)

                wait_gathers(p)

                @pl.when(i >= 1)
                def _():
                    wait_scatter()

                compute(p, K, contrib, trow[p], srow[p], erow[p])
                for j2 in range(K // 16):
                    gsc[pl.ds(j2 * 16, 16)] = gbv[p][pl.ds(j2 * 16, 16)]
                pltpu.async_copy(contrib, acc_sh.at[gsc], sem_s, add=True)
            return carry

        lax.fori_loop(0, NCH // 2, pair, 0)
        wait_scatter()

        tbase = ebase + NCH * K
        pltpu.sync_copy(a_hbm.at[pl.ds(tbase, TAILK)], avt)
        pltpu.sync_copy(b_hbm.at[pl.ds(tbase, TAILK)], bvt)
        for j in range(TAILK // 16):
            gat[pl.ds(j * 16, 16)] = avt[pl.ds(j * 16, 16)] * 2 + c
        pltpu.async_copy(prim_hbm.at[gat], trow[0].at[pl.ds(0, TAILK)],
                         sem_a[0]).wait()
        pltpu.async_copy(sec_hbm.at[bvt], srow[0].at[pl.ds(0, TAILK)],
                         sem_b[0]).wait()
        pltpu.async_copy(er_hbm.at[pl.ds(c * E + tbase, TAILK)],
                         erow[0].at[pl.ds(0, TAILK)], sem_e[0]).wait()
        compute(0, TAILK, contrib, trow[0], srow[0], erow[0])
        pltpu.sync_copy(contrib.at[pl.ds(0, TAILK)], acc_sh.at[bvt], add=True)

        plsc.subcore_barrier()
        pltpu.sync_copy(acc_sh.at[pl.ds(rbase, RPT)],
                        out_hbm.at[pl.ds(c * N + rbase, RPT)])

        @pl.when(s == NSUB - 1)
        def _():
            pltpu.sync_copy(acc_sh.at[pl.ds(NSUB * RPT, TAIL)],
                            out_hbm.at[pl.ds(c * N + NSUB * RPT, TAIL)])

    run_phase(src_hbm, dst_hbm, tf_hbm, b2_hbm, b3_hbm, outf_hbm)
    run_phase(dst_hbm, src_hbm, tb_hbm, c2_hbm, c3_hbm, outb_hbm)


def _sc_edge(src, dst, tf, b2t, tb, c2t, b3, c3, zeros):
    f = pl.kernel(
        _sc_edge_body,
        out_type=[
            jax.ShapeDtypeStruct((2 * N, D), jnp.float32),
            jax.ShapeDtypeStruct((2 * N, D), jnp.float32),
        ],
        mesh=plsc.VectorSubcoreMesh(core_axis_name="c", subcore_axis_name="s"),
        scratch_types=(
            [pltpu.VMEM((K,), jnp.int32)] * 9
            + [pltpu.VMEM((TAILK,), jnp.int32)] * 3
            + [pltpu.VMEM((K, D), jnp.float32)] * 4
            + [pltpu.VMEM((K, H), jnp.float32)] * 2
            + [pltpu.VMEM((K, D), jnp.float32)]
            + [pltpu.VMEM_SHARED((N, D), jnp.float32)]
            + [pltpu.SemaphoreType.DMA] * 11
        ),
    )
    return f(src, dst, tf, b2t, tb, c2t, b3, c3, zeros)


def kernel(h, edge_index, e_f, e_b, params):
    src = edge_index[0]
    dst = edge_index[1]

    names = ["A1", "A2", "A3", "B1", "B2", "C1", "C2"]
    wall = jnp.concatenate([params[n + "_w"].T for n in names], axis=1)
    ball = jnp.concatenate([params[n + "_b"] for n in names])[None, :]
    res = _node_mm(h, wall, ball)
    a1h = res[:, 0:D]
    a2h = res[:, D:2 * D]
    a3h = res[:, 2 * D:3 * D]
    b1h = res[:, 3 * D:4 * D]
    b2h = res[:, 4 * D:5 * D]
    c1h = res[:, 5 * D:6 * D]
    c2h = res[:, 6 * D:7 * D]

    tf = jnp.concatenate(
        [-b1h.reshape(N, 2, H), a2h.reshape(N, 2, H)], axis=2).reshape(2 * N, D)
    tb = jnp.concatenate(
        [-c1h.reshape(N, 2, H), a3h.reshape(N, 2, H)], axis=2).reshape(2 * N, D)
    b3 = _edge_mm(e_f, -params["B3_w"].T, -params["B3_b"][None, :])
    c3 = _edge_mm(e_b, -params["C3_w"].T, -params["C3_b"][None, :])
    b3 = b3.reshape(2 * E, H)
    c3 = c3.reshape(2 * E, H)

    zeros = jnp.zeros((RPT, D), jnp.float32)
    accf, accb = _sc_edge(src, dst, tf, -b2h, tb, -c2h, b3, c3, zeros)

    hpre, stats = _fin_a(a1h, accf, accb)
    h_new = _fin_b(hpre, stats, h, params["bn_h_g"][None, :],
                   params["bn_h_b"][None, :])
    return (h_new, e_f, e_b)


# speedup vs baseline: 1.0202x; 1.0202x over previous
import functools

import jax
import jax.numpy as jnp
from jax import lax
from jax.experimental import pallas as pl
from jax.experimental.pallas import tpu as pltpu
from jax.experimental.pallas import tpu_sc as plsc

N = 10000
E = 320000
D = 128
H = 64
K = 48
NSUB = 16
EPT = E // NSUB
NCH = EPT // K
TAILK = EPT - NCH * K
RPT = 624
TAIL = N - NSUB * RPT

RN = 1000
RE = 2000


def _node_mm_body(h_ref, w_ref, b_ref, a1_ref, tf_ref, tb_ref,
                  b2_ref, c2_ref):
    res = (
        jnp.dot(h_ref[...], w_ref[...], preferred_element_type=jnp.float32)
        + b_ref[...]
    )
    a2 = res[:, D:2 * D]
    a3 = res[:, 2 * D:3 * D]
    nb1 = res[:, 3 * D:4 * D]
    nc1 = res[:, 5 * D:6 * D]
    a1_ref[...] = res[:, 0:D]
    b2_ref[...] = res[:, 4 * D:5 * D]
    c2_ref[...] = res[:, 6 * D:7 * D]
    for cc in (0, 1):
        tf_ref[:, cc, 0:H] = nb1[:, cc * H:(cc + 1) * H]
        tf_ref[:, cc, H:D] = a2[:, cc * H:(cc + 1) * H]
        tb_ref[:, cc, 0:H] = nc1[:, cc * H:(cc + 1) * H]
        tb_ref[:, cc, H:D] = a3[:, cc * H:(cc + 1) * H]


def _node_mm(h, w, b):
    nd = jax.ShapeDtypeStruct((N, D), jnp.float32)
    n2d = jax.ShapeDtypeStruct((N, 2, D), jnp.float32)
    return pl.pallas_call(
        _node_mm_body,
        grid=(N // RN,),
        in_specs=[
            pl.BlockSpec((RN, D), lambda i: (i, 0)),
            pl.BlockSpec((D, 7 * D), lambda i: (0, 0)),
            pl.BlockSpec((1, 7 * D), lambda i: (0, 0)),
        ],
        out_specs=[
            pl.BlockSpec((RN, D), lambda i: (i, 0)),
            pl.BlockSpec((RN, 2, D), lambda i: (i, 0, 0)),
            pl.BlockSpec((RN, 2, D), lambda i: (i, 0, 0)),
            pl.BlockSpec((RN, D), lambda i: (i, 0)),
            pl.BlockSpec((RN, D), lambda i: (i, 0)),
        ],
        out_shape=[nd, n2d, n2d, nd, nd],
    )(h, w, b)


def _edge_mm_body(x_ref, w_ref, b_ref, o_ref):
    r = (
        jnp.dot(x_ref[...], w_ref[...], preferred_element_type=jnp.float32)
        + b_ref[...]
    )
    o_ref[0] = r[:, :H]
    o_ref[1] = r[:, H:]


def _edge_mm(x, w, b):
    return pl.pallas_call(
        _edge_mm_body,
        grid=(E // RE,),
        in_specs=[
            pl.BlockSpec((RE, D), lambda i: (i, 0)),
            pl.BlockSpec((D, D), lambda i: (0, 0)),
            pl.BlockSpec((1, D), lambda i: (0, 0)),
        ],
        out_specs=pl.BlockSpec((2, RE, H), lambda i: (0, i, 0)),
        out_shape=jax.ShapeDtypeStruct((2, E, H), jnp.float32),
    )(x, w, b)


def _fin_body(a1_ref, flo_ref, fhi_ref, blo_ref, bhi_ref, h_ref,
              g_ref, bb_ref, o_ref, hpre_s, ssum, ssq):
    p = pl.program_id(0)
    i = pl.program_id(1)

    @pl.when(p == 0)
    def _():
        nf = jnp.concatenate([flo_ref[:, :H], fhi_ref[:, :H]], axis=1)
        df = jnp.concatenate([flo_ref[:, H:], fhi_ref[:, H:]], axis=1)
        nb = jnp.concatenate([blo_ref[:, :H], bhi_ref[:, :H]], axis=1)
        db = jnp.concatenate([blo_ref[:, H:], bhi_ref[:, H:]], axis=1)
        hp = a1_ref[...] + nf / (df + 1e-6) + nb / (db + 1e-6)
        hpre_s[pl.ds(i * RN, RN), :] = hp

        @pl.when(i == 0)
        def _():
            ssum[...] = jnp.zeros_like(ssum)
            ssq[...] = jnp.zeros_like(ssq)

        ssum[...] += jnp.sum(hp, axis=0, keepdims=True)
        ssq[...] += jnp.sum(hp * hp, axis=0, keepdims=True)

    @pl.when(p == 1)
    def _():
        m = ssum[...] * (1.0 / N)
        v = ssq[...] * (1.0 / N) - m * m
        hp = hpre_s[pl.ds(i * RN, RN), :]
        y = g_ref[...] * (hp - m) / jnp.sqrt(v + 1e-5) + bb_ref[...]
        o_ref[...] = jnp.maximum(y, 0.0) + h_ref[...]


def _finalize(a1, accf, accb, h, g, b):
    nb_ = N // RN
    return pl.pallas_call(
        _fin_body,
        grid=(2, nb_),
        in_specs=[
            pl.BlockSpec((RN, D), lambda p, i: (i, 0)),
            pl.BlockSpec((RN, D), lambda p, i: (i, 0)),
            pl.BlockSpec((RN, D), lambda p, i: (nb_ + i, 0)),
            pl.BlockSpec((RN, D), lambda p, i: (i, 0)),
            pl.BlockSpec((RN, D), lambda p, i: (nb_ + i, 0)),
            pl.BlockSpec((RN, D), lambda p, i: (i, 0)),
            pl.BlockSpec((1, D), lambda p, i: (0, 0)),
            pl.BlockSpec((1, D), lambda p, i: (0, 0)),
        ],
        out_specs=pl.BlockSpec((RN, D), lambda p, i: (i, 0)),
        out_shape=jax.ShapeDtypeStruct((N, D), jnp.float32),
        scratch_shapes=[
            pltpu.VMEM((N, D), jnp.float32),
            pltpu.VMEM((1, D), jnp.float32),
            pltpu.VMEM((1, D), jnp.float32),
        ],
    )(a1, accf, accf, accb, accb, h, g, b)


def _sc_edge_body(src_hbm, dst_hbm, tf_hbm, b2_hbm, tb_hbm, c2_hbm,
                  b3_hbm, c3_hbm, zeros_hbm,
                  outf_hbm, outb_hbm,
                  av0, av1, bv0, bv1, ga0, ga1, gbv0, gbv1, gsc,
                  avt, bvt, gat,
                  trow0, trow1, srow0, srow1, erow0, erow1,
                  contrib, acc_sh,
                  sem_ia0, sem_ia1, sem_ib0, sem_ib1,
                  sem_a0, sem_a1, sem_b0, sem_b1, sem_e0, sem_e1, sem_s):
    av = (av0, av1)
    bv = (bv0, bv1)
    ga = (ga0, ga1)
    gbv = (gbv0, gbv1)
    trow = (trow0, trow1)
    srow = (srow0, srow1)
    erow = (erow0, erow1)
    sem_ia = (sem_ia0, sem_ia1)
    sem_ib = (sem_ib0, sem_ib1)
    sem_a = (sem_a0, sem_a1)
    sem_b = (sem_b0, sem_b1)
    sem_e = (sem_e0, sem_e1)
    c = lax.axis_index("c")
    s = lax.axis_index("s")
    ebase = s * EPT
    rbase = s * RPT

    def run_phase(a_hbm, b_hbm, prim_hbm, sec_hbm, er_hbm, out_hbm):
        pltpu.sync_copy(zeros_hbm.at[pl.ds(0, RPT)], acc_sh.at[pl.ds(rbase, RPT)])

        @pl.when(s == NSUB - 1)
        def _():
            pltpu.sync_copy(zeros_hbm.at[pl.ds(0, TAIL)],
                            acc_sh.at[pl.ds(NSUB * RPT, TAIL)])

        plsc.subcore_barrier()

        def issue_idx(i, slot):
            base = ebase + i * K
            pltpu.async_copy(a_hbm.at[pl.ds(base, K)], av[slot], sem_ia[slot])
            pltpu.async_copy(b_hbm.at[pl.ds(base, K)], bv[slot], sem_ib[slot])

        def wait_idx(slot):
            pltpu.make_async_copy(a_hbm.at[pl.ds(0, K)], av[slot],
                                  sem_ia[slot]).wait()
            pltpu.make_async_copy(b_hbm.at[pl.ds(0, K)], bv[slot],
                                  sem_ib[slot]).wait()

        def transform(slot):
            for j in range(K // 16):
                ga[slot][pl.ds(j * 16, 16)] = (
                    av[slot][pl.ds(j * 16, 16)] * 2 + c)
                gbv[slot][pl.ds(j * 16, 16)] = bv[slot][pl.ds(j * 16, 16)]

        def issue_gathers(i, slot):
            base = ebase + i * K
            pltpu.async_copy(prim_hbm.at[ga[slot]], trow[slot], sem_a[slot])
            pltpu.async_copy(sec_hbm.at[gbv[slot]], srow[slot], sem_b[slot])
            pltpu.async_copy(er_hbm.at[pl.ds(c * E + base, K)],
                             erow[slot], sem_e[slot])

        def wait_gathers(slot):
            pltpu.make_async_copy(prim_hbm.at[ga[slot]], trow[slot],
                                  sem_a[slot]).wait()
            pltpu.make_async_copy(sec_hbm.at[gbv[slot]], srow[slot],
                                  sem_b[slot]).wait()
            pltpu.make_async_copy(er_hbm.at[pl.ds(0, K)], erow[slot],
                                  sem_e[slot]).wait()

        def wait_scatter():
            pltpu.make_async_copy(contrib, acc_sh.at[gsc], sem_s).wait()

        def compute(slot, nk, cbuf, tr, sr, er):
            @plsc.parallel_loop(0, nk, unroll=4)
            def edge(e):
                for f in range(4):
                    b1 = tr[e, pl.ds(16 * f, 16)]
                    a2 = tr[e, pl.ds(H + 16 * f, 16)]
                    b2 = sr[e, pl.ds(c * H + 16 * f, 16)]
                    b3 = er[e, pl.ds(16 * f, 16)]
                    xn = b1 + b2 + b3
                    sg = 1.0 / (1.0 + jnp.exp(xn))
                    cbuf[e, pl.ds(16 * f, 16)] = a2 * sg
                    cbuf[e, pl.ds(H + 16 * f, 16)] = sg

        pltpu.sync_copy(a_hbm.at[pl.ds(ebase, K)], av[0])
        pltpu.sync_copy(b_hbm.at[pl.ds(ebase, K)], bv[0])
        transform(0)
        issue_gathers(0, 0)
        issue_idx(1, 1)

        def pair(j, carry):
            for p in (0, 1):
                i = 2 * j + p
                q = 1 - p

                @pl.when(i + 1 < NCH)
                def _():
                    wait_idx(q)
                    transform(q)
                    issue_gathers(i + 1, q)

                @pl.when(i + 2 < NCH)
                def _():
                    issue_idx(i + 2, p)

                wait_gathers(p)

                @pl.when(i >= 1)
                def _():
                    wait_scatter()

                compute(p, K, contrib, trow[p], srow[p], erow[p])
                for j2 in range(K // 16):
                    gsc[pl.ds(j2 * 16, 16)] = gbv[p][pl.ds(j2 * 16, 16)]
                pltpu.async_copy(contrib, acc_sh.at[gsc], sem_s, add=True)
            return carry

        lax.fori_loop(0, NCH // 2, pair, 0)
        wait_scatter()

        tbase = ebase + NCH * K
        pltpu.sync_copy(a_hbm.at[pl.ds(tbase, TAILK)], avt)
        pltpu.sync_copy(b_hbm.at[pl.ds(tbase, TAILK)], bvt)
        for j in range(TAILK // 16):
            gat[pl.ds(j * 16, 16)] = avt[pl.ds(j * 16, 16)] * 2 + c
        pltpu.async_copy(prim_hbm.at[gat], trow[0].at[pl.ds(0, TAILK)],
                         sem_a[0]).wait()
        pltpu.async_copy(sec_hbm.at[bvt], srow[0].at[pl.ds(0, TAILK)],
                         sem_b[0]).wait()
        pltpu.async_copy(er_hbm.at[pl.ds(c * E + tbase, TAILK)],
                         erow[0].at[pl.ds(0, TAILK)], sem_e[0]).wait()
        compute(0, TAILK, contrib, trow[0], srow[0], erow[0])
        pltpu.sync_copy(contrib.at[pl.ds(0, TAILK)], acc_sh.at[bvt], add=True)

        plsc.subcore_barrier()
        pltpu.sync_copy(acc_sh.at[pl.ds(rbase, RPT)],
                        out_hbm.at[pl.ds(c * N + rbase, RPT)])

        @pl.when(s == NSUB - 1)
        def _():
            pltpu.sync_copy(acc_sh.at[pl.ds(NSUB * RPT, TAIL)],
                            out_hbm.at[pl.ds(c * N + NSUB * RPT, TAIL)])

    run_phase(src_hbm, dst_hbm, tf_hbm, b2_hbm, b3_hbm, outf_hbm)
    run_phase(dst_hbm, src_hbm, tb_hbm, c2_hbm, c3_hbm, outb_hbm)


def _sc_edge(src, dst, tf, b2t, tb, c2t, b3, c3, zeros):
    f = pl.kernel(
        _sc_edge_body,
        out_type=[
            jax.ShapeDtypeStruct((2 * N, D), jnp.float32),
            jax.ShapeDtypeStruct((2 * N, D), jnp.float32),
        ],
        mesh=plsc.VectorSubcoreMesh(core_axis_name="c", subcore_axis_name="s"),
        scratch_types=(
            [pltpu.VMEM((K,), jnp.int32)] * 9
            + [pltpu.VMEM((TAILK,), jnp.int32)] * 3
            + [pltpu.VMEM((K, D), jnp.float32)] * 4
            + [pltpu.VMEM((K, H), jnp.float32)] * 2
            + [pltpu.VMEM((K, D), jnp.float32)]
            + [pltpu.VMEM_SHARED((N, D), jnp.float32)]
            + [pltpu.SemaphoreType.DMA] * 11
        ),
    )
    return f(src, dst, tf, b2t, tb, c2t, b3, c3, zeros)


def kernel(h, edge_index, e_f, e_b, params):
    src = edge_index[0]
    dst = edge_index[1]

    wall = jnp.concatenate(
        [params["A1_w"].T, params["A2_w"].T, params["A3_w"].T,
         -params["B1_w"].T, -params["B2_w"].T,
         -params["C1_w"].T, -params["C2_w"].T], axis=1)
    ball = jnp.concatenate(
        [params["A1_b"], params["A2_b"], params["A3_b"],
         -params["B1_b"], -params["B2_b"],
         -params["C1_b"], -params["C2_b"]])[None, :]
    a1h, tf3, tb3, b2n, c2n = _node_mm(h, wall, ball)
    tf = tf3.reshape(2 * N, D)
    tb = tb3.reshape(2 * N, D)

    b3 = _edge_mm(e_f, -params["B3_w"].T, -params["B3_b"][None, :])
    c3 = _edge_mm(e_b, -params["C3_w"].T, -params["C3_b"][None, :])
    b3 = b3.reshape(2 * E, H)
    c3 = c3.reshape(2 * E, H)

    zeros = jnp.zeros((RPT, D), jnp.float32)
    accf, accb = _sc_edge(src, dst, tf, b2n, tb, c2n, b3, c3, zeros)

    h_new = _finalize(a1h, accf, accb, h, params["bn_h_g"][None, :],
                      params["bn_h_b"][None, :])
    return (h_new, e_f, e_b)


# speedup vs baseline: 1.0875x; 1.0659x over previous
import functools

import jax
import jax.numpy as jnp
from jax import lax
from jax.experimental import pallas as pl
from jax.experimental.pallas import tpu as pltpu
from jax.experimental.pallas import tpu_sc as plsc

N = 10000
E = 320000
D = 128
H = 64
K = 48
NSUB = 16
EPT = E // NSUB
NCH = EPT // K
TAILK = EPT - NCH * K
RPT = 624
TAIL = N - NSUB * RPT

RN = 1000
RE = 2000


def _node_mm_body(h_ref, w_ref, b_ref, a1_ref, tf_ref, tb_ref,
                  b2_ref, c2_ref):
    res = (
        jnp.dot(h_ref[...], w_ref[...], preferred_element_type=jnp.float32)
        + b_ref[...]
    )
    a2 = res[:, D:2 * D]
    a3 = res[:, 2 * D:3 * D]
    nb1 = res[:, 3 * D:4 * D]
    nc1 = res[:, 5 * D:6 * D]
    a1_ref[...] = res[:, 0:D]
    b2_ref[...] = res[:, 4 * D:5 * D]
    c2_ref[...] = res[:, 6 * D:7 * D]
    for cc in (0, 1):
        tf_ref[:, cc, 0:H] = nb1[:, cc * H:(cc + 1) * H]
        tf_ref[:, cc, H:D] = a2[:, cc * H:(cc + 1) * H]
        tb_ref[:, cc, 0:H] = nc1[:, cc * H:(cc + 1) * H]
        tb_ref[:, cc, H:D] = a3[:, cc * H:(cc + 1) * H]


def _node_mm(h, w, b):
    nd = jax.ShapeDtypeStruct((N, D), jnp.float32)
    n2d = jax.ShapeDtypeStruct((N, 2, D), jnp.float32)
    return pl.pallas_call(
        _node_mm_body,
        grid=(N // RN,),
        in_specs=[
            pl.BlockSpec((RN, D), lambda i: (i, 0)),
            pl.BlockSpec((D, 7 * D), lambda i: (0, 0)),
            pl.BlockSpec((1, 7 * D), lambda i: (0, 0)),
        ],
        out_specs=[
            pl.BlockSpec((RN, D), lambda i: (i, 0)),
            pl.BlockSpec((RN, 2, D), lambda i: (i, 0, 0)),
            pl.BlockSpec((RN, 2, D), lambda i: (i, 0, 0)),
            pl.BlockSpec((RN, D), lambda i: (i, 0)),
            pl.BlockSpec((RN, D), lambda i: (i, 0)),
        ],
        out_shape=[nd, n2d, n2d, nd, nd],
    )(h, w, b)


def _edge_mm_body(x_ref, w_ref, b_ref, o_ref):
    r = (
        jnp.dot(x_ref[...], w_ref[...], preferred_element_type=jnp.float32)
        + b_ref[...]
    )
    o_ref[0] = r[:, :H]
    o_ref[1] = r[:, H:]


def _edge_mm(x, w, b):
    return pl.pallas_call(
        _edge_mm_body,
        grid=(E // RE,),
        in_specs=[
            pl.BlockSpec((RE, D), lambda i: (i, 0)),
            pl.BlockSpec((D, D), lambda i: (0, 0)),
            pl.BlockSpec((1, D), lambda i: (0, 0)),
        ],
        out_specs=pl.BlockSpec((2, RE, H), lambda i: (0, i, 0)),
        out_shape=jax.ShapeDtypeStruct((2, E, H), jnp.float32),
    )(x, w, b)


def _fin_body(a1_ref, flo_ref, fhi_ref, blo_ref, bhi_ref, h_ref,
              g_ref, bb_ref, o_ref, hpre_s, ssum, ssq):
    p = pl.program_id(0)
    i = pl.program_id(1)

    @pl.when(p == 0)
    def _():
        nf = jnp.concatenate([flo_ref[:, :H], fhi_ref[:, :H]], axis=1)
        df = jnp.concatenate([flo_ref[:, H:], fhi_ref[:, H:]], axis=1)
        nb = jnp.concatenate([blo_ref[:, :H], bhi_ref[:, :H]], axis=1)
        db = jnp.concatenate([blo_ref[:, H:], bhi_ref[:, H:]], axis=1)
        hp = a1_ref[...] + nf / (df + 1e-6) + nb / (db + 1e-6)
        hpre_s[pl.ds(i * RN, RN), :] = hp

        @pl.when(i == 0)
        def _():
            ssum[...] = jnp.zeros_like(ssum)
            ssq[...] = jnp.zeros_like(ssq)

        ssum[...] += jnp.sum(hp, axis=0, keepdims=True)
        ssq[...] += jnp.sum(hp * hp, axis=0, keepdims=True)

    @pl.when(p == 1)
    def _():
        m = ssum[...] * (1.0 / N)
        v = ssq[...] * (1.0 / N) - m * m
        hp = hpre_s[pl.ds(i * RN, RN), :]
        y = g_ref[...] * (hp - m) / jnp.sqrt(v + 1e-5) + bb_ref[...]
        o_ref[...] = jnp.maximum(y, 0.0) + h_ref[...]


def _finalize(a1, accf, accb, h, g, b):
    nb_ = N // RN
    return pl.pallas_call(
        _fin_body,
        grid=(2, nb_),
        in_specs=[
            pl.BlockSpec((RN, D), lambda p, i: (i, 0)),
            pl.BlockSpec((RN, D), lambda p, i: (i, 0)),
            pl.BlockSpec((RN, D), lambda p, i: (nb_ + i, 0)),
            pl.BlockSpec((RN, D), lambda p, i: (i, 0)),
            pl.BlockSpec((RN, D), lambda p, i: (nb_ + i, 0)),
            pl.BlockSpec((RN, D), lambda p, i: (i, 0)),
            pl.BlockSpec((1, D), lambda p, i: (0, 0)),
            pl.BlockSpec((1, D), lambda p, i: (0, 0)),
        ],
        out_specs=pl.BlockSpec((RN, D), lambda p, i: (i, 0)),
        out_shape=jax.ShapeDtypeStruct((N, D), jnp.float32),
        scratch_shapes=[
            pltpu.VMEM((N, D), jnp.float32),
            pltpu.VMEM((1, D), jnp.float32),
            pltpu.VMEM((1, D), jnp.float32),
        ],
    )(a1, accf, accf, accb, accb, h, g, b)


def _sc_edge_body(a_hbm, b_hbm, prim_hbm, sec_hbm, er_hbm, zeros_hbm,
                  out_hbm,
                  av0, av1, bv0, bv1, ga0, ga1, gbv0, gbv1, gsc,
                  avt, bvt, gat,
                  trow0, trow1, srow0, srow1, erow0, erow1,
                  contrib, acc_sh,
                  sem_ia0, sem_ia1, sem_ib0, sem_ib1,
                  sem_a0, sem_a1, sem_b0, sem_b1, sem_e0, sem_e1, sem_s):
    av = (av0, av1)
    bv = (bv0, bv1)
    ga = (ga0, ga1)
    gbv = (gbv0, gbv1)
    trow = (trow0, trow1)
    srow = (srow0, srow1)
    erow = (erow0, erow1)
    sem_ia = (sem_ia0, sem_ia1)
    sem_ib = (sem_ib0, sem_ib1)
    sem_a = (sem_a0, sem_a1)
    sem_b = (sem_b0, sem_b1)
    sem_e = (sem_e0, sem_e1)
    c = lax.axis_index("c")
    s = lax.axis_index("s")
    ebase = s * EPT
    rbase = s * RPT

    if True:
        pltpu.sync_copy(zeros_hbm.at[pl.ds(0, RPT)], acc_sh.at[pl.ds(rbase, RPT)])

        @pl.when(s == NSUB - 1)
        def _():
            pltpu.sync_copy(zeros_hbm.at[pl.ds(0, TAIL)],
                            acc_sh.at[pl.ds(NSUB * RPT, TAIL)])

        plsc.subcore_barrier()

        def issue_idx(i, slot):
            base = ebase + i * K
            pltpu.async_copy(a_hbm.at[pl.ds(base, K)], av[slot], sem_ia[slot])
            pltpu.async_copy(b_hbm.at[pl.ds(base, K)], bv[slot], sem_ib[slot])

        def wait_idx(slot):
            pltpu.make_async_copy(a_hbm.at[pl.ds(0, K)], av[slot],
                                  sem_ia[slot]).wait()
            pltpu.make_async_copy(b_hbm.at[pl.ds(0, K)], bv[slot],
                                  sem_ib[slot]).wait()

        def transform(slot):
            for j in range(K // 16):
                ga[slot][pl.ds(j * 16, 16)] = (
                    av[slot][pl.ds(j * 16, 16)] * 2 + c)
                gbv[slot][pl.ds(j * 16, 16)] = bv[slot][pl.ds(j * 16, 16)]

        def issue_gathers(i, slot):
            base = ebase + i * K
            pltpu.async_copy(prim_hbm.at[ga[slot]], trow[slot], sem_a[slot])
            pltpu.async_copy(sec_hbm.at[gbv[slot]], srow[slot], sem_b[slot])
            pltpu.async_copy(er_hbm.at[pl.ds(c * E + base, K)],
                             erow[slot], sem_e[slot])

        def wait_gathers(slot):
            pltpu.make_async_copy(prim_hbm.at[ga[slot]], trow[slot],
                                  sem_a[slot]).wait()
            pltpu.make_async_copy(sec_hbm.at[gbv[slot]], srow[slot],
                                  sem_b[slot]).wait()
            pltpu.make_async_copy(er_hbm.at[pl.ds(0, K)], erow[slot],
                                  sem_e[slot]).wait()

        def wait_scatter():
            pltpu.make_async_copy(contrib, acc_sh.at[gsc], sem_s).wait()

        def compute(slot, nk, cbuf, tr, sr, er):
            @plsc.parallel_loop(0, nk, unroll=4)
            def edge(e):
                for f in range(4):
                    b1 = tr[e, pl.ds(16 * f, 16)]
                    a2 = tr[e, pl.ds(H + 16 * f, 16)]
                    b2 = sr[e, pl.ds(c * H + 16 * f, 16)]
                    b3 = er[e, pl.ds(16 * f, 16)]
                    xn = b1 + b2 + b3
                    sg = 1.0 / (1.0 + jnp.exp(xn))
                    cbuf[e, pl.ds(16 * f, 16)] = a2 * sg
                    cbuf[e, pl.ds(H + 16 * f, 16)] = sg

        pltpu.sync_copy(a_hbm.at[pl.ds(ebase, K)], av[0])
        pltpu.sync_copy(b_hbm.at[pl.ds(ebase, K)], bv[0])
        transform(0)
        issue_gathers(0, 0)
        issue_idx(1, 1)

        def pair(j, carry):
            for p in (0, 1):
                i = 2 * j + p
                q = 1 - p

                @pl.when(i + 1 < NCH)
                def _():
                    wait_idx(q)
                    transform(q)
                    issue_gathers(i + 1, q)

                @pl.when(i + 2 < NCH)
                def _():
                    issue_idx(i + 2, p)

                wait_gathers(p)

                @pl.when(i >= 1)
                def _():
                    wait_scatter()

                compute(p, K, contrib, trow[p], srow[p], erow[p])
                for j2 in range(K // 16):
                    gsc[pl.ds(j2 * 16, 16)] = gbv[p][pl.ds(j2 * 16, 16)]
                pltpu.async_copy(contrib, acc_sh.at[gsc], sem_s, add=True)
            return carry

        lax.fori_loop(0, NCH // 2, pair, 0)
        wait_scatter()

        tbase = ebase + NCH * K
        pltpu.sync_copy(a_hbm.at[pl.ds(tbase, TAILK)], avt)
        pltpu.sync_copy(b_hbm.at[pl.ds(tbase, TAILK)], bvt)
        for j in range(TAILK // 16):
            gat[pl.ds(j * 16, 16)] = avt[pl.ds(j * 16, 16)] * 2 + c
        pltpu.async_copy(prim_hbm.at[gat], trow[0].at[pl.ds(0, TAILK)],
                         sem_a[0]).wait()
        pltpu.async_copy(sec_hbm.at[bvt], srow[0].at[pl.ds(0, TAILK)],
                         sem_b[0]).wait()
        pltpu.async_copy(er_hbm.at[pl.ds(c * E + tbase, TAILK)],
                         erow[0].at[pl.ds(0, TAILK)], sem_e[0]).wait()
        compute(0, TAILK, contrib, trow[0], srow[0], erow[0])
        pltpu.sync_copy(contrib.at[pl.ds(0, TAILK)], acc_sh.at[bvt], add=True)

        plsc.subcore_barrier()
        pltpu.sync_copy(acc_sh.at[pl.ds(rbase, RPT)],
                        out_hbm.at[pl.ds(c * N + rbase, RPT)])

        @pl.when(s == NSUB - 1)
        def _():
            pltpu.sync_copy(acc_sh.at[pl.ds(NSUB * RPT, TAIL)],
                            out_hbm.at[pl.ds(c * N + NSUB * RPT, TAIL)])


def _sc_edge_phase(a_idx, b_idx, prim, sec, er, zeros):
    f = pl.kernel(
        _sc_edge_body,
        out_type=jax.ShapeDtypeStruct((2 * N, D), jnp.float32),
        mesh=plsc.VectorSubcoreMesh(core_axis_name="c", subcore_axis_name="s"),
        scratch_types=(
            [pltpu.VMEM((K,), jnp.int32)] * 9
            + [pltpu.VMEM((TAILK,), jnp.int32)] * 3
            + [pltpu.VMEM((K, D), jnp.float32)] * 4
            + [pltpu.VMEM((K, H), jnp.float32)] * 2
            + [pltpu.VMEM((K, D), jnp.float32)]
            + [pltpu.VMEM_SHARED((N, D), jnp.float32)]
            + [pltpu.SemaphoreType.DMA] * 11
        ),
    )
    return f(a_idx, b_idx, prim, sec, er, zeros)


def kernel(h, edge_index, e_f, e_b, params):
    src = edge_index[0]
    dst = edge_index[1]

    wall = jnp.concatenate(
        [params["A1_w"].T, params["A2_w"].T, params["A3_w"].T,
         -params["B1_w"].T, -params["B2_w"].T,
         -params["C1_w"].T, -params["C2_w"].T], axis=1)
    ball = jnp.concatenate(
        [params["A1_b"], params["A2_b"], params["A3_b"],
         -params["B1_b"], -params["B2_b"],
         -params["C1_b"], -params["C2_b"]])[None, :]
    a1h, tf3, tb3, b2n, c2n = _node_mm(h, wall, ball)
    tf = tf3.reshape(2 * N, D)
    tb = tb3.reshape(2 * N, D)

    b3 = _edge_mm(e_f, -params["B3_w"].T, -params["B3_b"][None, :])
    c3 = _edge_mm(e_b, -params["C3_w"].T, -params["C3_b"][None, :])
    b3 = b3.reshape(2 * E, H)
    c3 = c3.reshape(2 * E, H)

    zeros = jnp.zeros((RPT, D), jnp.float32)
    accf = _sc_edge_phase(src, dst, tf, b2n, b3, zeros)
    accb = _sc_edge_phase(dst, src, tb, c2n, c3, zeros)

    h_new = _finalize(a1h, accf, accb, h, params["bn_h_g"][None, :],
                      params["bn_h_b"][None, :])
    return (h_new, e_f, e_b)
